# baseline (speedup 1.0000x reference)
import jax
import jax.numpy as jnp
from jax.experimental import pallas as pl

N = 10000
M = 50
EMBED = 16
IN_C = 64
HID = 64
OUT_C = 64
T = 4
CH = 128
NCH = (N + CH - 1) // CH


def _f32(x):
    return x.astype(jnp.float32)


def _dotT(a, b):
    return jax.lax.dot_general(a, b, (((1,), (1,)), ((), ())),
                               preferred_element_type=jnp.float32)


def _excl_cumsum_R(R):
    a_i = jax.lax.broadcasted_iota(jnp.int32, (CH, CH), 0)
    b_i = jax.lax.broadcasted_iota(jnp.int32, (CH, CH), 1)
    Texc = _f32(a_i < b_i)
    within = jnp.dot(R, Texc, preferred_element_type=jnp.float32)
    chunksum = jnp.sum(R, axis=1, keepdims=True)
    ai = jax.lax.broadcasted_iota(jnp.int32, (NCH, NCH), 0)
    bi = jax.lax.broadcasted_iota(jnp.int32, (NCH, NCH), 1)
    Aexc = _f32(bi < ai)
    chunk_excl = jnp.dot(Aexc, chunksum, preferred_element_type=jnp.float32)
    return within + chunk_excl


def _pre_body(nv_ref, ev_ref, maskTB_ref, maskDT_ref):
    DE = jnp.tanh(2.0 * nv_ref[...])
    EE = jnp.tanh(2.0 * ev_ref[...])
    logitsT = _dotT(EE, DE)
    adjT = jax.nn.relu(jnp.tanh(2.0 * logitsT))
    maskT = _f32(adjT > 0)
    cnt_row = jnp.sum(maskT, axis=0, keepdims=True)

    iT_c = jax.lax.broadcasted_iota(jnp.int32, (NCH, N), 0)
    iT_n = jax.lax.broadcasted_iota(jnp.int32, (NCH, N), 1)
    CmT = _f32(iT_n // CH == iT_c)
    q_q = jax.lax.broadcasted_iota(jnp.int32, (CH, N), 0)
    q_n = jax.lax.broadcasted_iota(jnp.int32, (CH, N), 1)
    PT = _f32(q_n % CH == q_q)

    cntR = _dotT(CmT, PT * cnt_row)
    rowoffR = _excl_cumsum_R(cntR)
    selR = jnp.where((rowoffR < M) & (cntR > 0), 1.0, 0.0)
    rrankR = _excl_cumsum_R(selR)

    G1 = jax.lax.dot_general(rowoffR, CmT, (((0,), (0,)), ((), ())),
                             preferred_element_type=jnp.float32)
    rowoff_row = jnp.sum(PT * G1, axis=0, keepdims=True)
    G2 = jax.lax.dot_general(rrankR, CmT, (((0,), (0,)), ((), ())),
                             preferred_element_type=jnp.float32)
    rrank_row = jnp.sum(PT * G2, axis=0, keepdims=True)
    sel_row = jnp.where((rowoff_row < M) & (cnt_row > 0), 1.0, 0.0)

    j_col = _f32(jax.lax.broadcasted_iota(jnp.int32, (M, 1), 0))
    ohT = jnp.where((rrank_row == j_col) & (sel_row > 0), 1.0, 0.0)
    cval = _dotT(ohT, adjT)
    cmask = _dotT(ohT, maskT)
    coff = _dotT(ohT, rowoff_row)
    a_i = jax.lax.broadcasted_iota(jnp.int32, (M, M), 0)
    b_i = jax.lax.broadcasted_iota(jnp.int32, (M, M), 1)
    Uincl = _f32(a_i <= b_i)
    ccum = jnp.dot(cmask, Uincl, preferred_element_type=jnp.float32)
    crank = coff + ccum - 1.0

    r_i = _f32(jax.lax.broadcasted_iota(jnp.int32, (M, M, M), 0))
    contrib = jnp.where(
        (crank[None, :, :] == r_i) & (cmask[None, :, :] > 0),
        cval[None, :, :], 0.0)
    valsM = jnp.sum(jnp.sum(contrib, axis=2), axis=1, keepdims=True)

    D_row = jnp.sum(maskT * valsM, axis=0, keepdims=True)
    dinv_row = jnp.where(D_row > 0, 1.0 / D_row, 0.0)
    Bdeg = jnp.sum(maskT, axis=1, keepdims=True)
    binv = jnp.where(Bdeg > 0, 1.0 / Bdeg, 0.0)

    maskTB_ref[...] = maskT * binv
    maskDT_ref[...] = maskT * dinv_row


def _body(xf_ref, maskTB_ref, maskDT_ref, Wg_ref, bg_ref, g1_ref, b1_ref,
          Wc1_ref, Wc2_ref, bc_ref, g2_ref, b2_ref, Wr1_ref, Wr2_ref, br_ref,
          Wo_ref, bf_ref, out_ref):
    def gather_scatter(xW):
        e = jnp.dot(maskTB_ref[...], xW,
                    preferred_element_type=jnp.float32)
        return jax.lax.dot_general(maskDT_ref[...], e, (((0,), (0,)), ((), ())),
                                   preferred_element_type=jnp.float32)

    def ln(v, g, b):
        mu = jnp.mean(v, axis=1, keepdims=True)
        var = jnp.mean((v - mu) ** 2, axis=1, keepdims=True)
        return (v - mu) / jnp.sqrt(var + 1e-5) * g + b

    Wg = Wg_ref[...]; bg = bg_ref[...]
    g1 = g1_ref[...]; b1 = b1_ref[...]
    Wc1 = Wc1_ref[...]; Wc2 = Wc2_ref[...]; bc = bc_ref[...]
    g2 = g2_ref[...]; b2 = b2_ref[...]
    Wr1 = Wr1_ref[...]; Wr2 = Wr2_ref[...]; br = br_ref[...]
    Wo = Wo_ref[...]

    j_i = jax.lax.broadcasted_iota(jnp.int32, (T * IN_C, IN_C), 0)
    c_i = jax.lax.broadcasted_iota(jnp.int32, (T * IN_C, IN_C), 1)

    def step(t, state):
        S = _f32(j_i == T * c_i + t)
        cx = jnp.dot(xf_ref[...], S,
                     preferred_element_type=jnp.float32)
        ias = jnp.concatenate([cx, state], axis=1)
        zr = gather_scatter(_dotT(ias, Wg)) + bg
        zr = jax.nn.sigmoid(ln(jax.nn.relu(ias + zr), g1, b1))
        z = zr[:, :HID]
        r = zr[:, HID:]
        rs = r * state
        res_c = _dotT(cx, Wr1) + _dotT(rs, Wr2) + br
        hc = gather_scatter(_dotT(cx, Wc1) + _dotT(rs, Wc2)) + bc
        hc = jnp.tanh(ln(jax.nn.relu(res_c + hc), g2, b2))
        state = (1.0 - z) * state + z * hc
        Wfold = jnp.dot(S, Wo, preferred_element_type=jnp.float32)
        out_ref[...] = out_ref[...] + _dotT(state, Wfold)
        return state

    bf = bf_ref[...]
    out_ref[...] = jnp.broadcast_to(bf, (N, T * OUT_C))
    jax.lax.fori_loop(0, T, step, jnp.zeros((N, HID), jnp.float32))


@jax.jit
def kernel(x, nodevec, edgevec, W_gate, b_gate, ln1_g, ln1_b,
           W_cand, b_cand, ln2_g, ln2_b, W_res, b_res, W_conv, b_conv):
    xf = x.reshape(N, T * IN_C)
    row = lambda v: v.reshape(1, -1)
    maskTB, maskDT = pl.pallas_call(
        _pre_body,
        out_shape=[jax.ShapeDtypeStruct((M, N), jnp.float32),
                   jax.ShapeDtypeStruct((M, N), jnp.float32)],
    )(nodevec, edgevec)

    bias_flat = jnp.repeat(b_conv, T).reshape(1, -1)
    res = pl.pallas_call(
        _body,
        out_shape=jax.ShapeDtypeStruct((N, T * OUT_C), jnp.float32),
    )(xf, maskTB, maskDT, W_gate, row(b_gate), row(ln1_g), row(ln1_b),
      W_cand[:, :IN_C], W_cand[:, IN_C:], row(b_cand), row(ln2_g), row(ln2_b),
      W_res[:, :IN_C], W_res[:, IN_C:], row(b_res), W_conv, bias_flat)
    return res.reshape(1, N, OUT_C, T)


# speedup vs baseline: 1.7585x; 1.7585x over previous
import jax
import jax.numpy as jnp
from jax.experimental import pallas as pl
from jax.experimental.pallas import tpu as pltpu

N = 10000
M = 50
EMBED = 16
IN_C = 64
HID = 64
OUT_C = 64
T = 4
CH = 128
NCH = (N + CH - 1) // CH


def _f32(x):
    return x.astype(jnp.float32)


def _dotT(a, b):
    return jax.lax.dot_general(a, b, (((1,), (1,)), ((), ())),
                               preferred_element_type=jnp.float32)


def _excl_cumsum_R(R):
    a_i = jax.lax.broadcasted_iota(jnp.int32, (CH, CH), 0)
    b_i = jax.lax.broadcasted_iota(jnp.int32, (CH, CH), 1)
    Texc = _f32(a_i < b_i)
    within = jnp.dot(R, Texc, preferred_element_type=jnp.float32)
    chunksum = jnp.sum(R, axis=1, keepdims=True)
    ai = jax.lax.broadcasted_iota(jnp.int32, (NCH, NCH), 0)
    bi = jax.lax.broadcasted_iota(jnp.int32, (NCH, NCH), 1)
    Aexc = _f32(bi < ai)
    chunk_excl = jnp.dot(Aexc, chunksum, preferred_element_type=jnp.float32)
    return within + chunk_excl


def _pre_body(nv_ref, ev_ref, maskTB_ref, maskD_ref):
    DE = jnp.tanh(2.0 * nv_ref[...])
    EE = jnp.tanh(2.0 * ev_ref[...])
    logitsT = _dotT(EE, DE)
    adjT = jax.nn.relu(jnp.tanh(2.0 * logitsT))
    maskT = _f32(adjT > 0)
    cnt_row = jnp.sum(maskT, axis=0, keepdims=True)

    iT_c = jax.lax.broadcasted_iota(jnp.int32, (NCH, N), 0)
    iT_n = jax.lax.broadcasted_iota(jnp.int32, (NCH, N), 1)
    CmT = _f32(iT_n // CH == iT_c)
    q_q = jax.lax.broadcasted_iota(jnp.int32, (CH, N), 0)
    q_n = jax.lax.broadcasted_iota(jnp.int32, (CH, N), 1)
    PT = _f32(q_n % CH == q_q)

    cntR = _dotT(CmT, PT * cnt_row)
    rowoffR = _excl_cumsum_R(cntR)
    selR = jnp.where((rowoffR < M) & (cntR > 0), 1.0, 0.0)
    rrankR = _excl_cumsum_R(selR)

    G1 = jax.lax.dot_general(rowoffR, CmT, (((0,), (0,)), ((), ())),
                             preferred_element_type=jnp.float32)
    rowoff_row = jnp.sum(PT * G1, axis=0, keepdims=True)
    G2 = jax.lax.dot_general(rrankR, CmT, (((0,), (0,)), ((), ())),
                             preferred_element_type=jnp.float32)
    rrank_row = jnp.sum(PT * G2, axis=0, keepdims=True)
    sel_row = jnp.where((rowoff_row < M) & (cnt_row > 0), 1.0, 0.0)

    j_col = _f32(jax.lax.broadcasted_iota(jnp.int32, (M, 1), 0))
    ohT = jnp.where((rrank_row == j_col) & (sel_row > 0), 1.0, 0.0)
    cval = _dotT(ohT, adjT)
    cmask = _dotT(ohT, maskT)
    coff = _dotT(ohT, rowoff_row)
    a_i = jax.lax.broadcasted_iota(jnp.int32, (M, M), 0)
    b_i = jax.lax.broadcasted_iota(jnp.int32, (M, M), 1)
    Uincl = _f32(a_i <= b_i)
    ccum = jnp.dot(cmask, Uincl, preferred_element_type=jnp.float32)
    crank = coff + ccum - 1.0

    r_i = _f32(jax.lax.broadcasted_iota(jnp.int32, (M, M, M), 0))
    contrib = jnp.where(
        (crank[None, :, :] == r_i) & (cmask[None, :, :] > 0),
        cval[None, :, :], 0.0)
    valsM = jnp.sum(jnp.sum(contrib, axis=2), axis=1, keepdims=True)

    D_row = jnp.sum(maskT * valsM, axis=0, keepdims=True)
    dinv_row = jnp.where(D_row > 0, 1.0 / D_row, 0.0)
    Bdeg = jnp.sum(maskT, axis=1, keepdims=True)
    binv = jnp.where(Bdeg > 0, 1.0 / Bdeg, 0.0)

    maskTB_ref[...] = maskT * binv
    maskD_ref[...] = jnp.transpose(maskT * dinv_row)


def _gru_body(xt_ref, maskTB_ref, maskD_ref, Wg_ref, bg_ref, g1_ref, b1_ref,
              Wc_ref, bc_ref, g2_ref, b2_ref, Wr_ref, br_ref, Wo_ref, bo_ref,
              out_ref, state_s):
    t = pl.program_id(0)

    @pl.when(t == 0)
    def _init():
        state_s[...] = jnp.zeros((N, HID), jnp.float32)

    maskTB = maskTB_ref[...]
    maskD = maskD_ref[...]
    state = state_s[...]
    cx = xt_ref[0]

    def hconv(v, W_ref, b_ref):
        xW = _dotT(v, W_ref[...])
        e = jnp.dot(maskTB, xW, preferred_element_type=jnp.float32)
        o = jnp.dot(maskD, e, preferred_element_type=jnp.float32)
        return o + b_ref[...]

    def ln(v, g_ref, b_ref):
        mu = jnp.mean(v, axis=1, keepdims=True)
        var = jnp.mean((v - mu) ** 2, axis=1, keepdims=True)
        return (v - mu) / jnp.sqrt(var + 1e-5) * g_ref[...] + b_ref[...]

    ias = jnp.concatenate([cx, state], axis=1)
    zr = hconv(ias, Wg_ref, bg_ref)
    zr = jax.nn.sigmoid(ln(jax.nn.relu(ias + zr), g1_ref, b1_ref))
    z = zr[:, :HID]
    r = zr[:, HID:]
    ci = jnp.concatenate([cx, r * state], axis=1)
    res_c = _dotT(ci, Wr_ref[...]) + br_ref[...]
    hc = hconv(ci, Wc_ref, bc_ref)
    hc = jnp.tanh(ln(jax.nn.relu(res_c + hc), g2_ref, b2_ref))
    new_state = (1.0 - z) * state + z * hc
    state_s[...] = new_state
    out_ref[0] = _dotT(new_state, Wo_ref[...]) + bo_ref[...]


def _full(shape):
    nd = len(shape)
    return pl.BlockSpec(shape, lambda t, _nd=nd: (0,) * _nd)


@jax.jit
def kernel(x, nodevec, edgevec, W_gate, b_gate, ln1_g, ln1_b,
           W_cand, b_cand, ln2_g, ln2_b, W_res, b_res, W_conv, b_conv):
    xt = jnp.transpose(x[0], (2, 0, 1))
    maskTB, maskD = pl.pallas_call(
        _pre_body,
        out_shape=[jax.ShapeDtypeStruct((M, N), jnp.float32),
                   jax.ShapeDtypeStruct((N, M), jnp.float32)],
    )(nodevec, edgevec)

    row = lambda v: v.reshape(1, -1)
    res = pl.pallas_call(
        _gru_body,
        grid=(T,),
        in_specs=[
            pl.BlockSpec((1, N, IN_C), lambda t: (t, 0, 0)),
            _full((M, N)),
            _full((N, M)),
            _full((2 * HID, IN_C + HID)),
            _full((1, 2 * HID)),
            _full((1, 2 * HID)),
            _full((1, 2 * HID)),
            _full((HID, IN_C + HID)),
            _full((1, HID)),
            _full((1, HID)),
            _full((1, HID)),
            _full((HID, IN_C + HID)),
            _full((1, HID)),
            _full((OUT_C, HID)),
            _full((1, OUT_C)),
        ],
        out_specs=pl.BlockSpec((1, N, OUT_C), lambda t: (t, 0, 0)),
        out_shape=jax.ShapeDtypeStruct((T, N, OUT_C), jnp.float32),
        scratch_shapes=[pltpu.VMEM((N, HID), jnp.float32)],
    )(xt, maskTB, maskD, W_gate, row(b_gate), row(ln1_g), row(ln1_b),
      W_cand, row(b_cand), row(ln2_g), row(ln2_b), W_res, row(b_res),
      W_conv, row(b_conv))
    return jnp.transpose(res, (1, 2, 0))[None]


# speedup vs baseline: 1.8813x; 1.0699x over previous
import jax
import jax.numpy as jnp
from jax.experimental import pallas as pl
from jax.experimental.pallas import tpu as pltpu

N = 10000
M = 50
EMBED = 16
IN_C = 64
HID = 64
OUT_C = 64
T = 4
CH = 128
NCH = (N + CH - 1) // CH


def _f32(x):
    return x.astype(jnp.float32)


def _dotT(a, b):
    return jax.lax.dot_general(a, b, (((1,), (1,)), ((), ())),
                               preferred_element_type=jnp.float32)


def _excl_cumsum_R(R):
    a_i = jax.lax.broadcasted_iota(jnp.int32, (CH, CH), 0)
    b_i = jax.lax.broadcasted_iota(jnp.int32, (CH, CH), 1)
    Texc = _f32(a_i < b_i)
    within = jnp.dot(R, Texc, preferred_element_type=jnp.float32)
    chunksum = jnp.sum(R, axis=1, keepdims=True)
    ai = jax.lax.broadcasted_iota(jnp.int32, (NCH, NCH), 0)
    bi = jax.lax.broadcasted_iota(jnp.int32, (NCH, NCH), 1)
    Aexc = _f32(bi < ai)
    chunk_excl = jnp.dot(Aexc, chunksum, preferred_element_type=jnp.float32)
    return within + chunk_excl


def _pre_body(nv_ref, ev_ref, maskTB_ref, maskD_ref):
    DE = jnp.tanh(2.0 * nv_ref[...])
    EE = jnp.tanh(2.0 * ev_ref[...])
    logitsT = _dotT(EE, DE)
    adjT = jax.nn.relu(jnp.tanh(2.0 * logitsT))
    maskT = _f32(adjT > 0)
    cnt_row = jnp.sum(maskT, axis=0, keepdims=True)

    iT_c = jax.lax.broadcasted_iota(jnp.int32, (NCH, N), 0)
    iT_n = jax.lax.broadcasted_iota(jnp.int32, (NCH, N), 1)
    CmT = _f32(iT_n // CH == iT_c)
    q_q = jax.lax.broadcasted_iota(jnp.int32, (CH, N), 0)
    q_n = jax.lax.broadcasted_iota(jnp.int32, (CH, N), 1)
    PT = _f32(q_n % CH == q_q)

    cntR = _dotT(CmT, PT * cnt_row)
    rowoffR = _excl_cumsum_R(cntR)
    selR = jnp.where((rowoffR < M) & (cntR > 0), 1.0, 0.0)
    rrankR = _excl_cumsum_R(selR)

    G1 = jax.lax.dot_general(rowoffR, CmT, (((0,), (0,)), ((), ())),
                             preferred_element_type=jnp.float32)
    rowoff_row = jnp.sum(PT * G1, axis=0, keepdims=True)
    G2 = jax.lax.dot_general(rrankR, CmT, (((0,), (0,)), ((), ())),
                             preferred_element_type=jnp.float32)
    rrank_row = jnp.sum(PT * G2, axis=0, keepdims=True)
    sel_row = jnp.where((rowoff_row < M) & (cnt_row > 0), 1.0, 0.0)

    j_col = _f32(jax.lax.broadcasted_iota(jnp.int32, (M, 1), 0))
    ohT = jnp.where((rrank_row == j_col) & (sel_row > 0), 1.0, 0.0)
    cval = _dotT(ohT, adjT)
    cmask = _dotT(ohT, maskT)
    coff = _dotT(ohT, rowoff_row)
    a_i = jax.lax.broadcasted_iota(jnp.int32, (M, M), 0)
    b_i = jax.lax.broadcasted_iota(jnp.int32, (M, M), 1)
    Uincl = _f32(a_i <= b_i)
    ccum = jnp.dot(cmask, Uincl, preferred_element_type=jnp.float32)
    crank = coff + ccum - 1.0

    r_i = _f32(jax.lax.broadcasted_iota(jnp.int32, (M, M, M), 0))
    contrib = jnp.where(
        (crank[None, :, :] == r_i) & (cmask[None, :, :] > 0),
        cval[None, :, :], 0.0)
    valsM = jnp.sum(jnp.sum(contrib, axis=2), axis=1, keepdims=True)

    D_row = jnp.sum(maskT * valsM, axis=0, keepdims=True)
    dinv_row = jnp.where(D_row > 0, 1.0 / D_row, 0.0)
    Bdeg = jnp.sum(maskT, axis=1, keepdims=True)
    binv = jnp.where(Bdeg > 0, 1.0 / Bdeg, 0.0)

    maskTB_ref[...] = maskT * binv
    maskD_ref[...] = jnp.transpose(maskT * dinv_row)


def _gru_body(xt_ref, maskTB_ref, maskD_ref, Wg_ref, bg_ref, g1_ref, b1_ref,
              Wc_ref, bc_ref, g2_ref, b2_ref, Wr_ref, br_ref, Wo_ref, bo_ref,
              out_ref, state_s):
    t = pl.program_id(0)

    @pl.when(t == 0)
    def _init():
        state_s[...] = jnp.zeros((N, HID), jnp.float32)

    maskTB = maskTB_ref[...]
    maskD = maskD_ref[...]
    state = state_s[...]
    cx = xt_ref[0]

    def hconv(v, W_ref, b_ref):
        xW = _dotT(v, W_ref[...])
        e = jnp.dot(maskTB, xW, preferred_element_type=jnp.float32)
        o = jnp.dot(maskD, e, preferred_element_type=jnp.float32)
        return o + b_ref[...]

    def ln(v, g_ref, b_ref):
        mu = jnp.mean(v, axis=1, keepdims=True)
        var = jnp.mean((v - mu) ** 2, axis=1, keepdims=True)
        return (v - mu) * jax.lax.rsqrt(var + 1e-5) * g_ref[...] + b_ref[...]

    ias = jnp.concatenate([cx, state], axis=1)
    zr = hconv(ias, Wg_ref, bg_ref)
    zr = jax.nn.sigmoid(ln(jax.nn.relu(ias + zr), g1_ref, b1_ref))
    z = zr[:, :HID]
    r = zr[:, HID:]
    ci = jnp.concatenate([cx, r * state], axis=1)
    res_c = _dotT(ci, Wr_ref[...]) + br_ref[...]
    hc = hconv(ci, Wc_ref, bc_ref)
    hc = jnp.tanh(ln(jax.nn.relu(res_c + hc), g2_ref, b2_ref))
    new_state = (1.0 - z) * state + z * hc
    state_s[...] = new_state
    out_ref[0] = _dotT(new_state, Wo_ref[...]) + bo_ref[...]


def _full(shape):
    nd = len(shape)
    return pl.BlockSpec(shape, lambda t, _nd=nd: (0,) * _nd)


@jax.jit
def kernel(x, nodevec, edgevec, W_gate, b_gate, ln1_g, ln1_b,
           W_cand, b_cand, ln2_g, ln2_b, W_res, b_res, W_conv, b_conv):
    xt = jnp.transpose(x[0], (2, 0, 1))
    maskTB, maskD = pl.pallas_call(
        _pre_body,
        out_shape=[jax.ShapeDtypeStruct((M, N), jnp.float32),
                   jax.ShapeDtypeStruct((N, M), jnp.float32)],
    )(nodevec, edgevec)

    row = lambda v: v.reshape(1, -1)
    res = pl.pallas_call(
        _gru_body,
        grid=(T,),
        in_specs=[
            pl.BlockSpec((1, N, IN_C), lambda t: (t, 0, 0)),
            _full((M, N)),
            _full((N, M)),
            _full((2 * HID, IN_C + HID)),
            _full((1, 2 * HID)),
            _full((1, 2 * HID)),
            _full((1, 2 * HID)),
            _full((HID, IN_C + HID)),
            _full((1, HID)),
            _full((1, HID)),
            _full((1, HID)),
            _full((HID, IN_C + HID)),
            _full((1, HID)),
            _full((OUT_C, HID)),
            _full((1, OUT_C)),
        ],
        out_specs=pl.BlockSpec((1, N, OUT_C), lambda t: (t, 0, 0)),
        out_shape=jax.ShapeDtypeStruct((T, N, OUT_C), jnp.float32),
        scratch_shapes=[pltpu.VMEM((N, HID), jnp.float32)],
    )(xt, maskTB, maskD, W_gate, row(b_gate), row(ln1_g), row(ln1_b),
      W_cand, row(b_cand), row(ln2_g), row(ln2_b), W_res, row(b_res),
      W_conv, row(b_conv))
    return jnp.transpose(res, (1, 2, 0))[None]
